# baseline (speedup 1.0000x reference)
import functools

import jax
import jax.numpy as jnp
from jax.experimental import pallas as pl
from jax.experimental.pallas import tpu as pltpu

_TOPK = 8


def _moe_body(x_ref, wg_ref, bg_ref, exp_ref, out_ref):
    x = x_ref[...]
    logits = (
        jnp.dot(x, wg_ref[...], preferred_element_type=jnp.float32)
        + bg_ref[...]
    )

    m = jnp.max(logits, axis=-1, keepdims=True)
    ex = jnp.exp(logits - m)
    w = ex / jnp.sum(ex, axis=-1, keepdims=True)

    t = logits
    neg = jnp.float32(-3.0e38)
    for _ in range(_TOPK - 1):
        mx = jnp.max(t, axis=-1, keepdims=True)
        t = jnp.where(t >= mx, neg, t)
    thresh = jnp.max(t, axis=-1, keepdims=True)
    wmask = jnp.where(logits >= thresh, w, 0.0)

    out_ref[...] = jnp.dot(
        wmask.astype(jnp.bfloat16),
        exp_ref[...].astype(jnp.bfloat16),
        preferred_element_type=jnp.float32,
    )


@jax.jit
def kernel(x, experts, Wg, bg):
    b, d = x.shape
    n_exp = experts.shape[0]
    tb = 512
    grid = (b // tb,)
    return pl.pallas_call(
        _moe_body,
        grid=grid,
        in_specs=[
            pl.BlockSpec((tb, d), lambda i: (i, 0)),
            pl.BlockSpec((d, n_exp), lambda i: (0, 0)),
            pl.BlockSpec((1, n_exp), lambda i: (0, 0)),
            pl.BlockSpec((n_exp, d), lambda i: (0, 0)),
        ],
        out_specs=pl.BlockSpec((tb, d), lambda i: (i, 0)),
        out_shape=jax.ShapeDtypeStruct((b, d), jnp.float32),
        compiler_params=pltpu.CompilerParams(
            dimension_semantics=("arbitrary",),
        ),
    )(x, Wg, bg.reshape(1, n_exp), experts)


# speedup vs baseline: 1.1286x; 1.1286x over previous
import functools

import jax
import jax.numpy as jnp
from jax.experimental import pallas as pl
from jax.experimental.pallas import tpu as pltpu

_TOPK = 8


def _moe_body(x_ref, wg_ref, bg_ref, exp_ref, out_ref, wm_ref):
    out_ref[...] = jnp.dot(
        wm_ref[...].astype(jnp.bfloat16),
        exp_ref[...].astype(jnp.bfloat16),
        preferred_element_type=jnp.float32,
    )

    x = x_ref[...]
    logits = (
        jnp.dot(x, wg_ref[...], preferred_element_type=jnp.float32)
        + bg_ref[...]
    )

    m = jnp.max(logits, axis=-1, keepdims=True)
    ex = jnp.exp(logits - m)
    w = ex / jnp.sum(ex, axis=-1, keepdims=True)

    t = logits
    neg = jnp.float32(-3.0e38)
    for _ in range(_TOPK - 1):
        mx = jnp.max(t, axis=-1, keepdims=True)
        t = jnp.where(t >= mx, neg, t)
    thresh = jnp.max(t, axis=-1, keepdims=True)
    wm_ref[...] = jnp.where(logits >= thresh, w, 0.0)


@jax.jit
def kernel(x, experts, Wg, bg):
    b, d = x.shape
    n_exp = experts.shape[0]
    tb = 512
    nb = b // tb
    return pl.pallas_call(
        _moe_body,
        grid=(nb + 1,),
        in_specs=[
            pl.BlockSpec((tb, d), lambda i: (jnp.minimum(i, nb - 1), 0)),
            pl.BlockSpec((d, n_exp), lambda i: (0, 0)),
            pl.BlockSpec((1, n_exp), lambda i: (0, 0)),
            pl.BlockSpec((n_exp, d), lambda i: (0, 0)),
        ],
        out_specs=pl.BlockSpec((tb, d), lambda i: (jnp.maximum(i - 1, 0), 0)),
        out_shape=jax.ShapeDtypeStruct((b, d), jnp.float32),
        scratch_shapes=[pltpu.VMEM((tb, n_exp), jnp.float32)],
        compiler_params=pltpu.CompilerParams(
            dimension_semantics=("arbitrary",),
        ),
    )(x, Wg, bg.reshape(1, n_exp), experts)


# speedup vs baseline: 1.1307x; 1.0019x over previous
import jax
import jax.numpy as jnp
from jax.experimental import pallas as pl
from jax.experimental.pallas import tpu as pltpu

_TOPK = 8


def _moe_body(x_ref, wg_ref, bg_ref, exp_ref, out_ref, wm_ref):
    out_ref[...] = jnp.dot(
        wm_ref[...].astype(jnp.bfloat16),
        exp_ref[...].astype(jnp.bfloat16),
        preferred_element_type=jnp.float32,
    )

    x = x_ref[...]
    logits = (
        jnp.dot(x, wg_ref[...], preferred_element_type=jnp.float32)
        + bg_ref[...]
    )

    m = jnp.max(logits, axis=-1, keepdims=True)
    ex = jnp.exp(logits - m)
    w = ex / jnp.sum(ex, axis=-1, keepdims=True)

    t = logits
    neg = jnp.float32(-3.0e38)
    for _ in range(_TOPK - 1):
        mx = jnp.max(t, axis=-1, keepdims=True)
        t = jnp.where(t >= mx, neg, t)
    thresh = jnp.max(t, axis=-1, keepdims=True)
    wm_ref[...] = jnp.where(logits >= thresh, w, 0.0)


@jax.jit
def kernel(x, experts, Wg, bg):
    b, d = x.shape
    n_exp = experts.shape[0]
    tb = 512
    nb = b // tb
    return pl.pallas_call(
        _moe_body,
        grid=(nb + 1,),
        in_specs=[
            pl.BlockSpec((tb, d), lambda i: (jnp.minimum(i, nb - 1), 0)),
            pl.BlockSpec((d, n_exp), lambda i: (0, 0)),
            pl.BlockSpec((1, n_exp), lambda i: (0, 0)),
            pl.BlockSpec((n_exp, d), lambda i: (0, 0)),
        ],
        out_specs=pl.BlockSpec((tb, d), lambda i: (jnp.maximum(i - 1, 0), 0)),
        out_shape=jax.ShapeDtypeStruct((b, d), jnp.float32),
        scratch_shapes=[pltpu.VMEM((tb, n_exp), jnp.float32)],
        compiler_params=pltpu.CompilerParams(
            dimension_semantics=("arbitrary",),
        ),
    )(x, Wg, bg.reshape(1, n_exp), experts)
